# Initial kernel scaffold; baseline (speedup 1.0000x reference)
import functools

import jax
import jax.numpy as jnp
from jax import lax
from jax.experimental import pallas as pl
from jax.experimental.pallas import tpu as pltpu
from jax.experimental.pallas import tpu_sc as plsc

NW = 32
L = 16
CHUNK = 128


def _normalize_group(buf, rows, d_dim):
    acc = jnp.zeros((L,), jnp.float32)
    for d in range(d_dim):
        col = jnp.full((L,), d, jnp.int32)
        v = plsc.load_gather(buf, [rows, col])
        acc = acc + v * v
    i = plsc.bitcast(acc, jnp.int32)
    i = jnp.int32(0x5F3759DF) - (i >> 1)
    y = plsc.bitcast(i, jnp.float32)
    for _ in range(3):
        y = y * (jnp.float32(1.5) - jnp.float32(0.5) * acc * y * y)
    for d in range(d_dim):
        col = jnp.full((L,), d, jnp.int32)
        v = plsc.load_gather(buf, [rows, col]) * y
        plsc.store_scatter(buf, [rows, col], v)


def kernel(indices, embeddings):
    b_shape = indices.shape
    d_dim = embeddings.shape[-1]
    flat_b = indices.size
    assert flat_b % (NW * CHUNK) == 0
    b_per_w = flat_b // NW
    n_chunks = b_per_w // CHUNK
    idx3 = indices.reshape(NW, n_chunks, CHUNK).astype(jnp.int32)

    mesh = plsc.VectorSubcoreMesh(core_axis_name="c", subcore_axis_name="s")

    @functools.partial(
        pl.kernel,
        mesh=mesh,
        out_type=jax.ShapeDtypeStruct((flat_b, d_dim), jnp.float32),
        scratch_types=[
            pltpu.VMEM((n_chunks, CHUNK), jnp.int32),
            pltpu.VMEM((CHUNK, d_dim), jnp.float32),
            pltpu.SemaphoreType.DMA,
        ],
    )
    def run(table_hbm, idx_hbm, out_hbm, idx_v, buf, sem):
        wid = lax.axis_index("s") * 2 + lax.axis_index("c")
        pltpu.sync_copy(idx_hbm.at[wid], idx_v)

        def chunk_body(c, carry):
            pltpu.async_copy(table_hbm.at[idx_v.at[c]], buf, sem).wait()
            for g in range(CHUNK // L):
                rows = jnp.arange(L, dtype=jnp.int32) + jnp.int32(g * L)
                _normalize_group(buf, rows, d_dim)
            base = wid * b_per_w + c * CHUNK
            pltpu.sync_copy(buf, out_hbm.at[pl.ds(base, CHUNK)])
            return carry

        lax.fori_loop(0, n_chunks, chunk_body, 0)

    out = run(embeddings, idx3)
    return out.reshape(*b_shape, d_dim)


# speedup vs baseline: 1.3993x; 1.3993x over previous
import functools

import jax
import jax.numpy as jnp
from jax import lax
from jax.experimental import pallas as pl
from jax.experimental.pallas import tpu as pltpu
from jax.experimental.pallas import tpu_sc as plsc

NW = 32
L = 16
ROWS_PER_CHUNK = 64


def _rsqrt(x):
    i = plsc.bitcast(x, jnp.int32)
    i = jnp.int32(0x5F3759DF) - (i >> 1)
    y = plsc.bitcast(i, jnp.float32)
    h = jnp.float32(0.5) * x
    for _ in range(3):
        y = y * (jnp.float32(1.5) - h * y * y)
    return y


def _normalize_chunk(buf):
    lanes = lax.iota(jnp.int32, L)
    for g in range(ROWS_PER_CHUNK // L):
        va = []
        vb = []
        sums = jnp.zeros((L,), jnp.float32)
        for r in range(L):
            row = g * L + r
            a = buf[2 * row]
            b = buf[2 * row + 1]
            va.append(a)
            vb.append(b)
            sq = a * a + b * b
            s = jnp.sum(sq)
            sums = jnp.where(lanes == r, s, sums)
        y = _rsqrt(sums)
        for r in range(L):
            row = g * L + r
            yr = jnp.take_along_axis(y, jnp.full((L,), r, jnp.int32),
                                     axis=0, mode="promise_in_bounds")
            buf[2 * row] = va[r] * yr
            buf[2 * row + 1] = vb[r] * yr


def kernel(indices, embeddings):
    b_shape = indices.shape
    d_dim = embeddings.shape[-1]
    assert d_dim == 2 * L
    flat_b = indices.size
    assert flat_b % (NW * ROWS_PER_CHUNK) == 0
    b_per_w = flat_b // NW
    n_chunks = b_per_w // ROWS_PER_CHUNK

    table2 = embeddings.reshape(-1, L)
    idx = indices.reshape(-1).astype(jnp.int32)
    idx2 = (2 * idx[:, None] + jnp.arange(2, dtype=jnp.int32)).reshape(
        NW, n_chunks, 2 * ROWS_PER_CHUNK)

    mesh = plsc.VectorSubcoreMesh(core_axis_name="c", subcore_axis_name="s")

    @functools.partial(
        pl.kernel,
        mesh=mesh,
        compiler_params=pltpu.CompilerParams(
            needs_layout_passes=False, use_tc_tiling_on_sc=False),
        out_type=jax.ShapeDtypeStruct((2 * flat_b, L), jnp.float32),
        scratch_types=[
            pltpu.VMEM((n_chunks, 2 * ROWS_PER_CHUNK), jnp.int32),
            pltpu.VMEM((2 * ROWS_PER_CHUNK, L), jnp.float32),
            pltpu.SemaphoreType.DMA,
        ],
    )
    def run(table_hbm, idx_hbm, out_hbm, idx_v, buf, sem):
        wid = lax.axis_index("s") * 2 + lax.axis_index("c")
        pltpu.sync_copy(idx_hbm.at[wid], idx_v)

        def chunk_body(c, carry):
            pltpu.async_copy(table_hbm.at[idx_v.at[c]], buf, sem).wait()
            _normalize_chunk(buf)
            base = 2 * (wid * b_per_w + c * ROWS_PER_CHUNK)
            pltpu.sync_copy(buf, out_hbm.at[pl.ds(base, 2 * ROWS_PER_CHUNK)])
            return carry

        lax.fori_loop(0, n_chunks, chunk_body, 0)

    out = run(table2, idx2)
    return out.reshape(*b_shape, d_dim)
